# baseline (speedup 1.0000x reference)
import functools

import jax
import jax.numpy as jnp
from jax import lax
from jax.experimental import pallas as pl
from jax.experimental.pallas import tpu as pltpu
from jax.experimental.pallas import tpu_sc as plsc

_INFO = plsc.get_sparse_core_info()
_NC = _INFO.num_cores
_NS = _INFO.num_subcores
_NW = _NC * _NS


def _make_sc_copy(seq_len, dim, dtype):
    rows_per_w = seq_len // _NW
    mesh = plsc.VectorSubcoreMesh(core_axis_name="c", subcore_axis_name="s")

    @functools.partial(
        pl.kernel,
        mesh=mesh,
        out_type=jax.ShapeDtypeStruct((seq_len, dim), dtype),
    )
    def sc_copy(emb_hbm, out_hbm):
        wid = lax.axis_index("c") * _NS + lax.axis_index("s")
        base = wid * rows_per_w
        pltpu.sync_copy(emb_hbm.at[pl.ds(base, rows_per_w)],
                        out_hbm.at[pl.ds(base, rows_per_w)])

    return sc_copy


def kernel(x, emb_table):
    seq_len = x.shape[1]
    dim = emb_table.shape[1]
    return _make_sc_copy(seq_len, dim, emb_table.dtype)(emb_table)


# speedup vs baseline: 24.1211x; 24.1211x over previous
import functools

import jax
import jax.numpy as jnp
from jax import lax
from jax.experimental import pallas as pl
from jax.experimental.pallas import tpu as pltpu
from jax.experimental.pallas import tpu_sc as plsc

_INFO = plsc.get_sparse_core_info()
_NC = _INFO.num_cores
_NS = _INFO.num_subcores
_NW = _NC * _NS

_CHUNK_ROWS = 32
_NBUF = 3


def _make_sc_copy(seq_len, dim, dtype):
    rows_per_w = seq_len // _NW
    n_chunks = rows_per_w // _CHUNK_ROWS
    mesh = plsc.VectorSubcoreMesh(core_axis_name="c", subcore_axis_name="s")

    scratch = [pltpu.VMEM((_CHUNK_ROWS, dim), dtype) for _ in range(_NBUF)]
    scratch += [pltpu.SemaphoreType.DMA for _ in range(2 * _NBUF)]

    @functools.partial(
        pl.kernel,
        mesh=mesh,
        out_type=jax.ShapeDtypeStruct((seq_len, dim), dtype),
        scratch_types=scratch,
    )
    def sc_copy(emb_hbm, out_hbm, *refs):
        bufs = refs[:_NBUF]
        in_sems = refs[_NBUF:2 * _NBUF]
        out_sems = refs[2 * _NBUF:]
        wid = lax.axis_index("c") * _NS + lax.axis_index("s")
        base = wid * rows_per_w

        def start_in(j):
            b = j % _NBUF
            return pltpu.async_copy(
                emb_hbm.at[pl.ds(base + j * _CHUNK_ROWS, _CHUNK_ROWS)],
                bufs[b], in_sems[b])

        def start_out(j):
            b = j % _NBUF
            return pltpu.async_copy(
                bufs[b],
                out_hbm.at[pl.ds(base + j * _CHUNK_ROWS, _CHUNK_ROWS)],
                out_sems[b])

        in_d = [None] * n_chunks
        out_d = [None] * n_chunks
        in_d[0] = start_in(0)
        for j in range(n_chunks):
            if j + 1 < n_chunks:
                if j + 1 >= _NBUF:
                    out_d[j + 1 - _NBUF].wait()
                in_d[j + 1] = start_in(j + 1)
            in_d[j].wait()
            out_d[j] = start_out(j)
        for j in range(max(0, n_chunks - _NBUF), n_chunks):
            out_d[j].wait()

    return sc_copy


def kernel(x, emb_table):
    seq_len = x.shape[1]
    dim = emb_table.shape[1]
    return _make_sc_copy(seq_len, dim, emb_table.dtype)(emb_table)
